# baseline (speedup 1.0000x reference)
import jax
import jax.numpy as jnp
from jax.experimental import pallas as pl
from jax.experimental.pallas import tpu as pltpu

_N = 16384
_NREAD = 4
_RQ = _N // _NREAD
_TC = 2048
_NT = _N // _TC
_NWBUF = 2


def _matmul_kernel(x_ref, mem_hbm, out_hbm, mv_ref, obufs, rsems, wsems):
    def read(q):
        return pltpu.make_async_copy(
            mem_hbm.at[pl.ds(q * _RQ, _RQ), :],
            mv_ref.at[pl.ds(q * _RQ, _RQ), :],
            rsems.at[q])

    def write(t):
        return pltpu.make_async_copy(
            obufs.at[t % _NWBUF],
            out_hbm.at[:, pl.ds(t * _TC, _TC)],
            wsems.at[t % _NWBUF])

    for q in range(_NREAD):
        read(q).start()

    tiles_per_q = _RQ // _TC
    for t in range(_NT):
        if t % tiles_per_q == 0:
            read(t // tiles_per_q).wait()
        if t >= _NWBUF:
            write(t - _NWBUF).wait()
        mb = mv_ref[pl.ds(t * _TC, _TC), :]
        obufs[t % _NWBUF] = jax.lax.dot_general(
            x_ref[...], mb, (((1,), (1,)), ((), ())),
            precision=jax.lax.Precision.DEFAULT,
            preferred_element_type=jnp.float32)
        write(t).start()

    for t in range(_NT - _NWBUF, _NT):
        write(t).wait()


def kernel(x, targets, memory):
    del targets
    b, d = x.shape
    n = memory.shape[0]
    return pl.pallas_call(
        _matmul_kernel,
        in_specs=[
            pl.BlockSpec((b, d), lambda: (0, 0)),
            pl.BlockSpec(memory_space=pltpu.MemorySpace.HBM),
        ],
        out_specs=pl.BlockSpec(memory_space=pltpu.MemorySpace.HBM),
        out_shape=jax.ShapeDtypeStruct((b, n), jnp.float32),
        scratch_shapes=[
            pltpu.VMEM((n, d), jnp.float32),
            pltpu.VMEM((_NWBUF, b, _TC), jnp.float32),
            pltpu.SemaphoreType.DMA((_NREAD,)),
            pltpu.SemaphoreType.DMA((_NWBUF,)),
        ],
    )(x, memory)


# speedup vs baseline: 1.0891x; 1.0891x over previous
import jax
import jax.numpy as jnp
from jax.experimental import pallas as pl
from jax.experimental.pallas import tpu as pltpu

_TILES = (1024, 2048, 2048, 2048, 2048, 2048, 2048, 2048, 1024)
_MAXT = max(_TILES)
_NBUF = 4


def _offsets(tiles):
    offs, o = [], 0
    for t in tiles:
        offs.append(o)
        o += t
    return tuple(offs)


_OFFS = _offsets(_TILES)


def _matmul_kernel(x_ref, mem_hbm, out_hbm, mbufs, obufs, rsems, wsems):
    nt = len(_TILES)

    def read(i):
        sz, off = _TILES[i], _OFFS[i]
        return pltpu.make_async_copy(
            mem_hbm.at[pl.ds(off, sz), :],
            mbufs.at[i % _NBUF, pl.ds(0, sz), :],
            rsems.at[i % _NBUF])

    def write(i):
        sz, off = _TILES[i], _OFFS[i]
        return pltpu.make_async_copy(
            obufs.at[i % _NBUF, :, pl.ds(0, sz)],
            out_hbm.at[:, pl.ds(off, sz)],
            wsems.at[i % _NBUF])

    for i in range(min(_NBUF, nt)):
        read(i).start()

    for i in range(nt):
        sz = _TILES[i]
        read(i).wait()
        if i >= _NBUF:
            write(i - _NBUF).wait()
        mb = mbufs[i % _NBUF, pl.ds(0, sz), :]
        obufs[i % _NBUF, :, pl.ds(0, sz)] = jax.lax.dot_general(
            x_ref[...], mb, (((1,), (1,)), ((), ())),
            precision=jax.lax.Precision.DEFAULT,
            preferred_element_type=jnp.float32)
        write(i).start()
        if i + _NBUF < nt:
            read(i + _NBUF).start()

    for i in range(max(nt - _NBUF, 0), nt):
        write(i).wait()


def kernel(x, targets, memory):
    del targets
    b, d = x.shape
    n = memory.shape[0]
    return pl.pallas_call(
        _matmul_kernel,
        in_specs=[
            pl.BlockSpec((b, d), lambda: (0, 0)),
            pl.BlockSpec(memory_space=pltpu.MemorySpace.HBM),
        ],
        out_specs=pl.BlockSpec(memory_space=pltpu.MemorySpace.HBM),
        out_shape=jax.ShapeDtypeStruct((b, n), jnp.float32),
        scratch_shapes=[
            pltpu.VMEM((_NBUF, _MAXT, d), jnp.float32),
            pltpu.VMEM((_NBUF, b, _MAXT), jnp.float32),
            pltpu.SemaphoreType.DMA((_NBUF,)),
            pltpu.SemaphoreType.DMA((_NBUF,)),
        ],
    )(x, memory)
